# baseline (speedup 1.0000x reference)
import functools

import jax
import jax.numpy as jnp
from jax.experimental import pallas as pl
from jax.experimental.pallas import tpu as pltpu

QUES = 2048
HID = 1024
T = 32768
K = 32

_INTERPRET = False


def _alpha_body(qblk_ref, qvec_ref, out_ref):
    out_ref[...] = jax.lax.dot_general(
        qvec_ref[...], qblk_ref[...],
        (((1,), (1,)), ((), ())),
        preferred_element_type=jnp.float32)[None]


def _alpha(questions, question):
    G = 32
    BT = T // G
    return pl.pallas_call(
        _alpha_body,
        grid=(G,),
        in_specs=[
            pl.BlockSpec((BT, QUES), lambda i: (i, 0)),
            pl.BlockSpec((1, QUES), lambda i: (0, 0)),
        ],
        out_specs=pl.BlockSpec((1, 1, BT), lambda i: (i, 0, 0)),
        out_shape=jax.ShapeDtypeStruct((G, 1, BT), jnp.float32),
        interpret=_INTERPRET,
    )(questions, question.reshape(1, QUES))


def _attn_body(alpha_ref, hs_ref, out_ref, rows_ref, sems):
    a = alpha_ref[...]
    r_i = jax.lax.broadcasted_iota(jnp.int32, a.shape, 0)
    c_i = jax.lax.broadcasted_iota(jnp.int32, a.shape, 1)
    flat = r_i * a.shape[1] + c_i
    neg = jnp.float32(-jnp.inf)
    big = jnp.int32(2**30)
    vals = []
    for t in range(K):
        m = jnp.max(a)
        eq = a == m
        fi = jnp.min(jnp.where(eq, flat, big))
        pltpu.make_async_copy(hs_ref.at[fi], rows_ref.at[t], sems.at[t]).start()
        vals.append(m)
        a = jnp.where(flat == fi, neg, a)
    m0 = vals[0]
    ws = [jnp.exp(v - m0) for v in vals]
    z = functools.reduce(lambda x, y: x + y, ws)
    acc = jnp.zeros((1, HID), dtype=jnp.float32)
    for t in range(K):
        pltpu.make_async_copy(hs_ref.at[0], rows_ref.at[t], sems.at[t]).wait()
        acc = acc + rows_ref[t][None, :] * (ws[t] / z)
    out_ref[...] = acc


def _attn(alpha2d, hs_flat):
    return pl.pallas_call(
        _attn_body,
        in_specs=[
            pl.BlockSpec(memory_space=pltpu.VMEM),
            pl.BlockSpec(memory_space=pl.ANY),
        ],
        out_specs=pl.BlockSpec(memory_space=pltpu.VMEM),
        out_shape=jax.ShapeDtypeStruct((1, HID), jnp.float32),
        scratch_shapes=[
            pltpu.VMEM((K, HID), jnp.float32),
            pltpu.SemaphoreType.DMA((K,)),
        ],
        interpret=_INTERPRET,
    )(alpha2d, hs_flat)


def _gru_body(q_ref, s_ref, attn_ref, h0r_ref, h0c_ref, wsc_ref, bsc_ref,
              wih_ref, whh_ref, bih_ref, bhh_ref, pred_ref, h_ref):
    j = pl.program_id(0)
    q = q_ref[...]
    s = s_ref[0, 0]
    m_ge = (s >= 0.5).astype(jnp.float32)
    m_lt = (s < 0.5).astype(jnp.float32)
    x = jnp.concatenate([q * m_ge, q * m_lt], axis=1)

    @pl.when(j == 0)
    def _():
        wsc = wsc_ref[...]
        pred = (jnp.sum(q * wsc[:, :QUES]) + jnp.sum(attn_ref[...] * wsc[:, QUES:])
                + bsc_ref[0, 0])
        pred_ref[0, 0] = pred

    gi3 = jax.lax.dot_general(
        wih_ref[...], x, (((2,), (1,)), ((), ())),
        preferred_element_type=jnp.float32)
    gh3 = jax.lax.dot_general(
        whh_ref[...], h0r_ref[...], (((2,), (1,)), ((), ())),
        preferred_element_type=jnp.float32)
    i_r, i_z, i_n = gi3[0] + bih_ref[0], gi3[1] + bih_ref[1], gi3[2] + bih_ref[2]
    h_r, h_z, h_n = gh3[0] + bhh_ref[0], gh3[1] + bhh_ref[1], gh3[2] + bhh_ref[2]
    r = jax.nn.sigmoid(i_r + h_r)
    z = jax.nn.sigmoid(i_z + h_z)
    n = jnp.tanh(i_n + r * h_n)
    h_ref[...] = (1.0 - z) * n + z * h0c_ref[...]


def _gru(question, score, attn, h0, W_score, b_score, W_ih, W_hh, b_ih, b_hh):
    Gj = 8
    Bh = HID // Gj
    wih3 = W_ih.reshape(3, HID, 2 * QUES)
    whh3 = W_hh.reshape(3, HID, HID)
    bih3 = b_ih.reshape(3, HID, 1)
    bhh3 = b_hh.reshape(3, HID, 1)
    pred, h_new = pl.pallas_call(
        _gru_body,
        grid=(Gj,),
        in_specs=[
            pl.BlockSpec((1, QUES), lambda j: (0, 0)),
            pl.BlockSpec((1, 1), lambda j: (0, 0), memory_space=pltpu.SMEM),
            pl.BlockSpec((1, HID), lambda j: (0, 0)),
            pl.BlockSpec((1, HID), lambda j: (0, 0)),
            pl.BlockSpec((Bh, 1), lambda j: (j, 0)),
            pl.BlockSpec((1, QUES + HID), lambda j: (0, 0)),
            pl.BlockSpec((1, 1), lambda j: (0, 0), memory_space=pltpu.SMEM),
            pl.BlockSpec((3, Bh, 2 * QUES), lambda j: (0, j, 0)),
            pl.BlockSpec((3, Bh, HID), lambda j: (0, j, 0)),
            pl.BlockSpec((3, Bh, 1), lambda j: (0, j, 0)),
            pl.BlockSpec((3, Bh, 1), lambda j: (0, j, 0)),
        ],
        out_specs=[
            pl.BlockSpec((1, 1), lambda j: (0, 0), memory_space=pltpu.SMEM),
            pl.BlockSpec((Bh, 1), lambda j: (j, 0)),
        ],
        out_shape=[
            jax.ShapeDtypeStruct((1, 1), jnp.float32),
            jax.ShapeDtypeStruct((HID, 1), jnp.float32),
        ],
        interpret=_INTERPRET,
    )(question.reshape(1, QUES), score.reshape(1, 1), attn,
      h0.reshape(1, HID), h0.reshape(HID, 1), W_score,
      b_score.reshape(1, 1), wih3, whh3, bih3, bhh3)
    return pred, h_new


def kernel(question, score, questions, hs, W_score, b_score, W_ih, W_hh, b_ih, b_hh):
    hs_flat = hs.reshape(T, HID)
    alpha2d = _alpha(questions, question).reshape(32, T // 32)
    attn = _attn(alpha2d, hs_flat)
    h0 = hs_flat[T - 1]
    pred, h_new = _gru(question, score, attn, h0, W_score, b_score,
                       W_ih, W_hh, b_ih, b_hh)
    return pred, h_new.reshape(1, 1, HID)


# speedup vs baseline: 1.7819x; 1.7819x over previous
import functools

import jax
import jax.numpy as jnp
from jax.experimental import pallas as pl
from jax.experimental.pallas import tpu as pltpu

QUES = 2048
HID = 1024
T = 32768
K = 32

_INTERPRET = False


def _alpha_body(qblk_ref, qvec_ref, out_ref):
    out_ref[...] = jax.lax.dot_general(
        qvec_ref[...], qblk_ref[...],
        (((1,), (1,)), ((), ())),
        preferred_element_type=jnp.float32)[None]


def _alpha(questions, question):
    G = 32
    BT = T // G
    return pl.pallas_call(
        _alpha_body,
        grid=(G,),
        in_specs=[
            pl.BlockSpec((BT, QUES), lambda i: (i, 0)),
            pl.BlockSpec((1, QUES), lambda i: (0, 0)),
        ],
        out_specs=pl.BlockSpec((1, 1, BT), lambda i: (i, 0, 0)),
        out_shape=jax.ShapeDtypeStruct((G, 1, BT), jnp.float32),
        interpret=_INTERPRET,
    )(questions, question.reshape(1, QUES))


def _attn_body(alpha_ref, hs_ref, out_ref, rows_ref, sems):
    a = alpha_ref[...]
    r_i = jax.lax.broadcasted_iota(jnp.int32, a.shape, 0)
    c_i = jax.lax.broadcasted_iota(jnp.int32, a.shape, 1)
    flat = r_i * a.shape[1] + c_i
    neg = jnp.float32(-jnp.inf)
    big = jnp.int32(2**30)
    vals = []
    for t in range(K):
        m = jnp.max(a)
        eq = a == m
        fi = jnp.min(jnp.where(eq, flat, big))
        pltpu.make_async_copy(hs_ref.at[fi], rows_ref.at[t], sems.at[t]).start()
        vals.append(m)
        a = jnp.where(flat == fi, neg, a)
    m0 = vals[0]
    ws = [jnp.exp(v - m0) for v in vals]
    z = functools.reduce(lambda x, y: x + y, ws)
    acc = jnp.zeros((1, HID), dtype=jnp.float32)
    for t in range(K):
        pltpu.make_async_copy(hs_ref.at[0], rows_ref.at[t], sems.at[t]).wait()
        acc = acc + rows_ref[t] * (ws[t] / z)
    out_ref[...] = acc


def _attn(alpha2d, hs):
    return pl.pallas_call(
        _attn_body,
        in_specs=[
            pl.BlockSpec(memory_space=pltpu.VMEM),
            pl.BlockSpec(memory_space=pl.ANY),
        ],
        out_specs=pl.BlockSpec(memory_space=pltpu.VMEM),
        out_shape=jax.ShapeDtypeStruct((1, HID), jnp.float32),
        scratch_shapes=[
            pltpu.VMEM((K, 1, HID), jnp.float32),
            pltpu.SemaphoreType.DMA((K,)),
        ],
        interpret=_INTERPRET,
    )(alpha2d, hs)


def _gru_body(q_ref, s_ref, attn_ref, h0r_ref, h0c_ref, wsc_ref, bsc_ref,
              wih_ref, whh_ref, bih_ref, bhh_ref, pred_ref, h_ref):
    j = pl.program_id(0)
    q = q_ref[...]
    s = s_ref[0, 0]
    m_ge = (s >= 0.5).astype(jnp.float32)
    m_lt = (s < 0.5).astype(jnp.float32)
    x = jnp.concatenate([q * m_ge, q * m_lt], axis=1)

    @pl.when(j == 0)
    def _():
        wsc = wsc_ref[...]
        pred = (jnp.sum(q * wsc[:, :QUES]) + jnp.sum(attn_ref[...] * wsc[:, QUES:])
                + bsc_ref[0, 0])
        pred_ref[0, 0] = pred

    gi3 = jax.lax.dot_general(
        wih_ref[...], x, (((2,), (1,)), ((), ())),
        preferred_element_type=jnp.float32)
    gh3 = jax.lax.dot_general(
        whh_ref[...], h0r_ref[...], (((2,), (1,)), ((), ())),
        preferred_element_type=jnp.float32)
    i_r, i_z, i_n = gi3[0] + bih_ref[0], gi3[1] + bih_ref[1], gi3[2] + bih_ref[2]
    h_r, h_z, h_n = gh3[0] + bhh_ref[0], gh3[1] + bhh_ref[1], gh3[2] + bhh_ref[2]
    r = jax.nn.sigmoid(i_r + h_r)
    z = jax.nn.sigmoid(i_z + h_z)
    n = jnp.tanh(i_n + r * h_n)
    h_ref[...] = (1.0 - z) * n + z * h0c_ref[...]


def _gru(question, score, attn, h0, W_score, b_score, W_ih, W_hh, b_ih, b_hh):
    Gj = 8
    Bh = HID // Gj
    wih3 = W_ih.reshape(3, HID, 2 * QUES)
    whh3 = W_hh.reshape(3, HID, HID)
    bih3 = b_ih.reshape(3, HID, 1)
    bhh3 = b_hh.reshape(3, HID, 1)
    pred, h_new = pl.pallas_call(
        _gru_body,
        grid=(Gj,),
        in_specs=[
            pl.BlockSpec((1, QUES), lambda j: (0, 0)),
            pl.BlockSpec((1, 1), lambda j: (0, 0), memory_space=pltpu.SMEM),
            pl.BlockSpec((1, HID), lambda j: (0, 0)),
            pl.BlockSpec((1, HID), lambda j: (0, 0)),
            pl.BlockSpec((Bh, 1), lambda j: (j, 0)),
            pl.BlockSpec((1, QUES + HID), lambda j: (0, 0)),
            pl.BlockSpec((1, 1), lambda j: (0, 0), memory_space=pltpu.SMEM),
            pl.BlockSpec((3, Bh, 2 * QUES), lambda j: (0, j, 0)),
            pl.BlockSpec((3, Bh, HID), lambda j: (0, j, 0)),
            pl.BlockSpec((3, Bh, 1), lambda j: (0, j, 0)),
            pl.BlockSpec((3, Bh, 1), lambda j: (0, j, 0)),
        ],
        out_specs=[
            pl.BlockSpec((1, 1), lambda j: (0, 0), memory_space=pltpu.SMEM),
            pl.BlockSpec((Bh, 1), lambda j: (j, 0)),
        ],
        out_shape=[
            jax.ShapeDtypeStruct((1, 1), jnp.float32),
            jax.ShapeDtypeStruct((HID, 1), jnp.float32),
        ],
        interpret=_INTERPRET,
    )(question.reshape(1, QUES), score.reshape(1, 1), attn,
      h0.reshape(1, HID), h0.reshape(HID, 1), W_score,
      b_score.reshape(1, 1), wih3, whh3, bih3, bhh3)
    return pred, h_new


def kernel(question, score, questions, hs, W_score, b_score, W_ih, W_hh, b_ih, b_hh):
    alpha2d = _alpha(questions, question).reshape(32, T // 32)
    attn = _attn(alpha2d, hs)
    h0 = hs[T - 1, 0]
    pred, h_new = _gru(question, score, attn, h0, W_score, b_score,
                       W_ih, W_hh, b_ih, b_hh)
    return pred, h_new.reshape(1, 1, HID)
